# baseline (speedup 1.0000x reference)
import functools

import numpy as np
import jax
import jax.numpy as jnp
from jax import lax
from jax.experimental import pallas as pl
from jax.experimental.pallas import tpu as pltpu
from jax.experimental.pallas import tpu_sc as plsc

_BATCH = 16384
_FIELDS = 26
_EMB = 16
_BF = _BATCH * _FIELDS
_NW = 32
_PER_W = _BF // _NW
_CH = 3328
_NCH = _PER_W // _CH
_FPAD = 32
_LANES = _FPAD * _EMB
_NT = _LANES // 128
_ROWS2 = _NT * _BATCH * 8
_BM = 512

_SLOT = np.arange(_BF, dtype=np.int64)
_DB = _SLOT // _FIELDS
_DF = _SLOT % _FIELDS
_DSTMAP = ((_DF >> 3) * (_BATCH * 8) + _DB * 8 + (_DF & 7)).astype(np.int32)


def _sc_gather(idx, dstmap, t2, t1):
    mesh = plsc.VectorSubcoreMesh(core_axis_name="c", subcore_axis_name="s")

    @functools.partial(
        pl.kernel,
        mesh=mesh,
        out_type=(
            jax.ShapeDtypeStruct((_ROWS2, _EMB), jnp.float32),
            jax.ShapeDtypeStruct((_BF,), jnp.float32),
        ),
        scratch_types=(
            pltpu.VMEM((_CH,), jnp.int32),
            pltpu.VMEM((_CH // 128, 128), jnp.int32),
            pltpu.VMEM((_CH, _EMB), jnp.float32),
            pltpu.VMEM((_CH,), jnp.float32),
            pltpu.SemaphoreType.DMA,
            pltpu.SemaphoreType.DMA,
        ),
        compiler_params=pltpu.CompilerParams(use_tc_tiling_on_sc=False),
    )
    def k(idx_hbm, dst_hbm, t2_hbm, t1_hbm, out2_hbm, out1_hbm,
          idx_v, dst_v, r2_v, r1_v, s2, s1):
        wid = lax.axis_index("s") * 2 + lax.axis_index("c")
        for c in range(_NCH):
            off = wid * _PER_W + c * _CH
            pltpu.sync_copy(idx_hbm.at[pl.ds(off, _CH)], idx_v)
            cp2 = pltpu.async_copy(t2_hbm.at[idx_v], r2_v, s2)
            cp1 = pltpu.async_copy(t1_hbm.at[idx_v], r1_v, s1)
            pltpu.sync_copy(dst_hbm.at[pl.ds(off // 128, _CH // 128)], dst_v)
            cp2.wait()
            cp1.wait()
            cps = [
                pltpu.async_copy(
                    r2_v.at[pl.ds(j * 128, 128)],
                    out2_hbm.at[dst_v.at[j]], s2)
                for j in range(_CH // 128)
            ]
            for cp in cps:
                cp.wait()
            pltpu.sync_copy(r1_v, out1_hbm.at[pl.ds(off, _CH)])

    return k(idx, dstmap, t2, t1)


_VOC = 1000000
_TRC = 2048


def _tr_body(tin, t1in, tout, t1out):
    tout[:] = tin[:].T
    t1out[:] = t1in[0, :]


def _tc_transpose(t2t, t1t):
    return pl.pallas_call(
        _tr_body,
        grid=(pl.cdiv(_VOC, _TRC),),
        in_specs=[pl.BlockSpec((_EMB, _TRC), lambda i: (0, i)),
                  pl.BlockSpec((1, _TRC), lambda i: (0, i))],
        out_specs=[pl.BlockSpec((_TRC, _EMB), lambda i: (i, 0)),
                   pl.BlockSpec((_TRC,), lambda i: (i,))],
        out_shape=[jax.ShapeDtypeStruct((_VOC, _EMB), jnp.float32),
                   jax.ShapeDtypeStruct((_VOC,), jnp.float32)],
    )(t2t, t1t)


def _tc_body(e2, e1, xv, em, sm, w1, b1, w2, b2, bz, out):
    lane = lax.broadcasted_iota(jnp.int32, (1, _LANES), 1)
    d = jnp.concatenate([e2[0], e2[1], e2[2], e2[3]], axis=1)
    xvx = jnp.dot(xv[:], em[:], preferred_element_type=jnp.float32)
    d = jnp.where(lane < _FIELDS * _EMB, d * xvx, 0.0)
    first = jnp.sum(e1[:] * xv[:], axis=1)
    s = jnp.dot(d, sm[:], preferred_element_type=jnp.float32)
    second = 0.5 * (jnp.sum(s * s, axis=1) - jnp.sum(d * d, axis=1))
    x = jnp.maximum(jnp.dot(d, w1[:], preferred_element_type=jnp.float32) + b1[:], 0.0)
    x = jnp.maximum(jnp.dot(x, w2[:], preferred_element_type=jnp.float32) + b2[:], 0.0)
    out[:] = first + second + jnp.sum(x, axis=1) + bz[0, 0]


def _tc_dense(e2, e1, xv, w1p, b1, w2, b2, bias):
    d1 = w1p.shape[1]
    d2 = w2.shape[1]
    em = np.zeros((_FIELDS, _LANES), np.float32)
    sm = np.zeros((_LANES, _EMB), np.float32)
    for f in range(_FIELDS):
        for e in range(_EMB):
            em[f, f * _EMB + e] = 1.0
            sm[f * _EMB + e, e] = 1.0
    em = jnp.asarray(em)
    sm = jnp.asarray(sm)
    return pl.pallas_call(
        _tc_body,
        grid=(_BATCH // _BM,),
        in_specs=[
            pl.BlockSpec((_NT, _BM, 128), lambda i: (0, i, 0)),
            pl.BlockSpec((_BM, _FIELDS), lambda i: (i, 0)),
            pl.BlockSpec((_BM, _FIELDS), lambda i: (i, 0)),
            pl.BlockSpec((_FIELDS, _LANES), lambda i: (0, 0)),
            pl.BlockSpec((_LANES, _EMB), lambda i: (0, 0)),
            pl.BlockSpec((_LANES, d1), lambda i: (0, 0)),
            pl.BlockSpec((1, d1), lambda i: (0, 0)),
            pl.BlockSpec((d1, d2), lambda i: (0, 0)),
            pl.BlockSpec((1, d2), lambda i: (0, 0)),
            pl.BlockSpec((1, 1), lambda i: (0, 0)),
        ],
        out_specs=pl.BlockSpec((_BM,), lambda i: (i,)),
        out_shape=jax.ShapeDtypeStruct((_BATCH,), jnp.float32),
    )(e2, e1, xv, em, sm, w1p, b1.reshape(1, d1), w2, b2.reshape(1, d2),
      bias.reshape(1, 1))


def kernel(Xi, Xv, fm_first_w, fm_second_w, W1, b1, W2, b2, bias):
    idx = Xi.reshape(_BF).astype(jnp.int32)
    t2r, t1r = _tc_transpose(fm_second_w.T, fm_first_w.T)
    emb2, emb1 = _sc_gather(idx, jnp.asarray(_DSTMAP.reshape(-1, 128)),
                            t2r, t1r)
    e2 = emb2.reshape(_NT, _BATCH, 128)
    e1 = emb1.reshape(_BATCH, _FIELDS)
    d1 = W1.shape[1]
    w1p = jnp.pad(W1.reshape(_FIELDS, _EMB, d1),
                  ((0, _FPAD - _FIELDS), (0, 0), (0, 0))).reshape(_LANES, d1)
    return _tc_dense(e2, e1, Xv, w1p, b1, W2, b2, bias)


# speedup vs baseline: 1.3530x; 1.3530x over previous
import functools

import numpy as np
import jax
import jax.numpy as jnp
from jax import lax
from jax.experimental import pallas as pl
from jax.experimental.pallas import tpu as pltpu
from jax.experimental.pallas import tpu_sc as plsc

_BATCH = 16384
_FIELDS = 26
_EMB = 16
_BF = _BATCH * _FIELDS
_NW = 32
_PER_W = _BF // _NW
_CH = 3328
_NCH = _PER_W // _CH
_FPAD = 32
_LANES = _FPAD * _EMB
_NT = _LANES // 128
_ROWS2 = _NT * _BATCH * 8
_BM = 512

_SLOT = np.arange(_BF, dtype=np.int64)
_DB = _SLOT // _FIELDS
_DF = _SLOT % _FIELDS
_DSTMAP = ((_DF >> 3) * (_BATCH * 8) + _DB * 8 + (_DF & 7)).astype(np.int32)


def _sc_gather(idx, dstmap, t2, t1):
    mesh = plsc.VectorSubcoreMesh(core_axis_name="c", subcore_axis_name="s")

    @functools.partial(
        pl.kernel,
        mesh=mesh,
        out_type=(
            jax.ShapeDtypeStruct((_ROWS2, _EMB), jnp.float32),
            jax.ShapeDtypeStruct((_BF,), jnp.float32),
        ),
        scratch_types=(
            pltpu.VMEM((_CH,), jnp.int32),
            pltpu.VMEM((_CH // 128, 128), jnp.int32),
            pltpu.VMEM((_CH, _EMB), jnp.float32),
            pltpu.VMEM((_CH,), jnp.float32),
            pltpu.SemaphoreType.DMA,
            pltpu.SemaphoreType.DMA,
        ),
        compiler_params=pltpu.CompilerParams(use_tc_tiling_on_sc=False),
    )
    def k(idx_hbm, dst_hbm, t2_hbm, t1_hbm, out2_hbm, out1_hbm,
          idx_v, dst_v, r2_v, r1_v, s2, s1):
        wid = lax.axis_index("s") * 2 + lax.axis_index("c")
        for c in range(_NCH):
            off = wid * _PER_W + c * _CH
            pltpu.sync_copy(idx_hbm.at[pl.ds(off, _CH)], idx_v)
            cp2 = pltpu.async_copy(t2_hbm.at[idx_v], r2_v, s2)
            cp1 = pltpu.async_copy(t1_hbm.at[idx_v], r1_v, s1)
            pltpu.sync_copy(dst_hbm.at[pl.ds(off // 128, _CH // 128)], dst_v)
            cp2.wait()
            cp1.wait()
            cps = [
                pltpu.async_copy(
                    r2_v.at[pl.ds(j * 128, 128)],
                    out2_hbm.at[dst_v.at[j]], s2)
                for j in range(_CH // 128)
            ]
            for cp in cps:
                cp.wait()
            pltpu.sync_copy(r1_v, out1_hbm.at[pl.ds(off, _CH)])

    return k(idx, dstmap, t2, t1)


_VOC = 1000000
_TRC = 16384


def _tr_body(tin, t1in, eye, tout, t1out):
    tout[:] = lax.dot_general(tin[:], eye[:], (((0,), (0,)), ((), ())),
                              preferred_element_type=jnp.float32)
    t1out[:] = t1in[0, :]


def _tc_transpose(t2t, t1t):
    return pl.pallas_call(
        _tr_body,
        grid=(pl.cdiv(_VOC, _TRC),),
        in_specs=[pl.BlockSpec((_EMB, _TRC), lambda i: (0, i)),
                  pl.BlockSpec((1, _TRC), lambda i: (0, i)),
                  pl.BlockSpec((_EMB, _EMB), lambda i: (0, 0))],
        out_specs=[pl.BlockSpec((_TRC, _EMB), lambda i: (i, 0)),
                   pl.BlockSpec((_TRC,), lambda i: (i,))],
        out_shape=[jax.ShapeDtypeStruct((_VOC, _EMB), jnp.float32),
                   jax.ShapeDtypeStruct((_VOC,), jnp.float32)],
    )(t2t, t1t, jnp.eye(_EMB, dtype=jnp.float32))


def _tc_body(e2, e1, xv, em, sm, w1, b1, w2, b2, bz, out):
    lane = lax.broadcasted_iota(jnp.int32, (1, _LANES), 1)
    d = jnp.concatenate([e2[0], e2[1], e2[2], e2[3]], axis=1)
    xvx = jnp.dot(xv[:], em[:], preferred_element_type=jnp.float32)
    d = jnp.where(lane < _FIELDS * _EMB, d * xvx, 0.0)
    first = jnp.sum(e1[:] * xv[:], axis=1)
    s = jnp.dot(d, sm[:], preferred_element_type=jnp.float32)
    second = 0.5 * (jnp.sum(s * s, axis=1) - jnp.sum(d * d, axis=1))
    x = jnp.maximum(jnp.dot(d, w1[:], preferred_element_type=jnp.float32) + b1[:], 0.0)
    x = jnp.maximum(jnp.dot(x, w2[:], preferred_element_type=jnp.float32) + b2[:], 0.0)
    out[:] = first + second + jnp.sum(x, axis=1) + bz[0, 0]


def _tc_dense(e2, e1, xv, w1p, b1, w2, b2, bias):
    d1 = w1p.shape[1]
    d2 = w2.shape[1]
    em = np.zeros((_FIELDS, _LANES), np.float32)
    sm = np.zeros((_LANES, _EMB), np.float32)
    for f in range(_FIELDS):
        for e in range(_EMB):
            em[f, f * _EMB + e] = 1.0
            sm[f * _EMB + e, e] = 1.0
    em = jnp.asarray(em)
    sm = jnp.asarray(sm)
    return pl.pallas_call(
        _tc_body,
        grid=(_BATCH // _BM,),
        in_specs=[
            pl.BlockSpec((_NT, _BM, 128), lambda i: (0, i, 0)),
            pl.BlockSpec((_BM, _FIELDS), lambda i: (i, 0)),
            pl.BlockSpec((_BM, _FIELDS), lambda i: (i, 0)),
            pl.BlockSpec((_FIELDS, _LANES), lambda i: (0, 0)),
            pl.BlockSpec((_LANES, _EMB), lambda i: (0, 0)),
            pl.BlockSpec((_LANES, d1), lambda i: (0, 0)),
            pl.BlockSpec((1, d1), lambda i: (0, 0)),
            pl.BlockSpec((d1, d2), lambda i: (0, 0)),
            pl.BlockSpec((1, d2), lambda i: (0, 0)),
            pl.BlockSpec((1, 1), lambda i: (0, 0)),
        ],
        out_specs=pl.BlockSpec((_BM,), lambda i: (i,)),
        out_shape=jax.ShapeDtypeStruct((_BATCH,), jnp.float32),
    )(e2, e1, xv, em, sm, w1p, b1.reshape(1, d1), w2, b2.reshape(1, d2),
      bias.reshape(1, 1))


def kernel(Xi, Xv, fm_first_w, fm_second_w, W1, b1, W2, b2, bias):
    idx = Xi.reshape(_BF).astype(jnp.int32)
    t2r, t1r = _tc_transpose(fm_second_w.T, fm_first_w.T)
    emb2, emb1 = _sc_gather(idx, jnp.asarray(_DSTMAP.reshape(-1, 128)),
                            t2r, t1r)
    e2 = emb2.reshape(_NT, _BATCH, 128)
    e1 = emb1.reshape(_BATCH, _FIELDS)
    d1 = W1.shape[1]
    w1p = jnp.pad(W1.reshape(_FIELDS, _EMB, d1),
                  ((0, _FPAD - _FIELDS), (0, 0), (0, 0))).reshape(_LANES, d1)
    return _tc_dense(e2, e1, Xv, w1p, b1, W2, b2, bias)


# speedup vs baseline: 1.5071x; 1.1139x over previous
import functools

import numpy as np
import jax
import jax.numpy as jnp
from jax import lax
from jax.experimental import pallas as pl
from jax.experimental.pallas import tpu as pltpu
from jax.experimental.pallas import tpu_sc as plsc

_BATCH = 16384
_FIELDS = 26
_EMB = 16
_BF = _BATCH * _FIELDS
_NW = 32
_PER_W = _BF // _NW
_CH = 3328
_NCH = _PER_W // _CH
_FPAD = 32
_LANES = _FPAD * _EMB
_NT = _LANES // 128
_ROWS2 = _NT * _BATCH * 8
_BM = 512

_SLOT = np.arange(_BF, dtype=np.int64)
_DB = _SLOT // _FIELDS
_DF = _SLOT % _FIELDS
_DSTMAP = ((_DF >> 3) * (_BATCH * 8) + _DB * 8 + (_DF & 7)).astype(np.int32)


def _sc_gather(idx, dstmap, t2, t1):
    mesh = plsc.VectorSubcoreMesh(core_axis_name="c", subcore_axis_name="s")

    @functools.partial(
        pl.kernel,
        mesh=mesh,
        out_type=(
            jax.ShapeDtypeStruct((_ROWS2, _EMB), jnp.float32),
            jax.ShapeDtypeStruct((_BF,), jnp.float32),
        ),
        scratch_types=(
            pltpu.VMEM((_CH,), jnp.int32),
            pltpu.VMEM((_CH // 128, 128), jnp.int32),
            pltpu.VMEM((_CH, _EMB), jnp.float32),
            pltpu.VMEM((_CH,), jnp.float32),
            pltpu.SemaphoreType.DMA,
            pltpu.SemaphoreType.DMA,
        ),
        compiler_params=pltpu.CompilerParams(use_tc_tiling_on_sc=False),
    )
    def k(idx_hbm, dst_hbm, t2_hbm, t1_hbm, out2_hbm, out1_hbm,
          idx_v, dst_v, r2_v, r1_v, s2, s1):
        wid = lax.axis_index("s") * 2 + lax.axis_index("c")
        for c in range(_NCH):
            off = wid * _PER_W + c * _CH
            pltpu.sync_copy(idx_hbm.at[pl.ds(off, _CH)], idx_v)
            cp2 = pltpu.async_copy(t2_hbm.at[idx_v], r2_v, s2)
            cp1 = pltpu.async_copy(t1_hbm.at[idx_v], r1_v, s1)
            pltpu.sync_copy(dst_hbm.at[pl.ds(off // 128, _CH // 128)], dst_v)
            cp2.wait()
            cp1.wait()
            cps = [
                pltpu.async_copy(
                    r2_v.at[pl.ds(j * 128, 128)],
                    out2_hbm.at[dst_v.at[j]], s2)
                for j in range(_CH // 128)
            ]
            for cp in cps:
                cp.wait()
            pltpu.sync_copy(r1_v, out1_hbm.at[pl.ds(off, _CH)])

    return k(idx, dstmap, t2, t1)


def _tc_body(e2, e1, xv, em, sm, w1, b1, w2, b2, bz, out):
    lane = lax.broadcasted_iota(jnp.int32, (1, _LANES), 1)
    d = jnp.concatenate([e2[0], e2[1], e2[2], e2[3]], axis=1)
    xvx = jnp.dot(xv[:], em[:], preferred_element_type=jnp.float32)
    d = jnp.where(lane < _FIELDS * _EMB, d * xvx, 0.0)
    first = jnp.sum(e1[:] * xv[:], axis=1)
    s = jnp.dot(d, sm[:], preferred_element_type=jnp.float32)
    second = 0.5 * (jnp.sum(s * s, axis=1) - jnp.sum(d * d, axis=1))
    x = jnp.maximum(jnp.dot(d, w1[:], preferred_element_type=jnp.float32) + b1[:], 0.0)
    x = jnp.maximum(jnp.dot(x, w2[:], preferred_element_type=jnp.float32) + b2[:], 0.0)
    out[:] = first + second + jnp.sum(x, axis=1) + bz[0, 0]


def _tc_dense(e2, e1, xv, w1p, b1, w2, b2, bias):
    d1 = w1p.shape[1]
    d2 = w2.shape[1]
    em = np.zeros((_FIELDS, _LANES), np.float32)
    sm = np.zeros((_LANES, _EMB), np.float32)
    for f in range(_FIELDS):
        for e in range(_EMB):
            em[f, f * _EMB + e] = 1.0
            sm[f * _EMB + e, e] = 1.0
    em = jnp.asarray(em)
    sm = jnp.asarray(sm)
    return pl.pallas_call(
        _tc_body,
        grid=(_BATCH // _BM,),
        in_specs=[
            pl.BlockSpec((_NT, _BM, 128), lambda i: (0, i, 0)),
            pl.BlockSpec((_BM, _FIELDS), lambda i: (i, 0)),
            pl.BlockSpec((_BM, _FIELDS), lambda i: (i, 0)),
            pl.BlockSpec((_FIELDS, _LANES), lambda i: (0, 0)),
            pl.BlockSpec((_LANES, _EMB), lambda i: (0, 0)),
            pl.BlockSpec((_LANES, d1), lambda i: (0, 0)),
            pl.BlockSpec((1, d1), lambda i: (0, 0)),
            pl.BlockSpec((d1, d2), lambda i: (0, 0)),
            pl.BlockSpec((1, d2), lambda i: (0, 0)),
            pl.BlockSpec((1, 1), lambda i: (0, 0)),
        ],
        out_specs=pl.BlockSpec((_BM,), lambda i: (i,)),
        out_shape=jax.ShapeDtypeStruct((_BATCH,), jnp.float32),
    )(e2, e1, xv, em, sm, w1p, b1.reshape(1, d1), w2, b2.reshape(1, d2),
      bias.reshape(1, 1))


def kernel(Xi, Xv, fm_first_w, fm_second_w, W1, b1, W2, b2, bias):
    idx = Xi.reshape(_BF).astype(jnp.int32)
    emb2, emb1 = _sc_gather(idx, jnp.asarray(_DSTMAP.reshape(-1, 128)),
                            fm_second_w, fm_first_w.reshape(-1))
    e2 = emb2.reshape(_NT, _BATCH, 128)
    e1 = emb1.reshape(_BATCH, _FIELDS)
    d1 = W1.shape[1]
    w1p = jnp.pad(W1.reshape(_FIELDS, _EMB, d1),
                  ((0, _FPAD - _FIELDS), (0, 0), (0, 0))).reshape(_LANES, d1)
    return _tc_dense(e2, e1, Xv, w1p, b1, W2, b2, bias)
